# Initial kernel scaffold; baseline (speedup 1.0000x reference)
import jax
import jax.numpy as jnp
from jax.experimental import pallas as pl


def kernel(x, table):
    raise NotImplementedError("write your pallas kernel here")


# speedup vs baseline: 1.7395x; 1.7395x over previous
import functools

import numpy as np
import jax
import jax.numpy as jnp
from jax import lax
from jax.experimental import pallas as pl
from jax.experimental.pallas import tpu as pltpu
from jax.experimental.pallas import tpu_sc as plsc

EMBED = 64
SEQ = 50
NUM_CORES = 2
NUM_SUBCORES = 16
NUM_WORKERS = NUM_CORES * NUM_SUBCORES
CHUNK = 100
LANES = 16
VREGS_PER_ROW = EMBED // LANES


def _positional_encoding(seq_len, d_model):
    pos = np.arange(seq_len)[:, np.newaxis]
    i = np.arange(d_model)[np.newaxis, :]
    angle_rates = 1.0 / np.power(10000, 2 * (i // 2) / np.float32(d_model))
    angle_rads = pos * angle_rates
    angle_rads[:, 0::2] = np.sin(angle_rads[:, 0::2])
    angle_rads[:, 1::2] = np.cos(angle_rads[:, 1::2])
    return angle_rads.astype(np.float32)


@functools.lru_cache(maxsize=None)
def _build(rows_per_worker, vocab):
    num_chunks = rows_per_worker // CHUNK

    @functools.partial(
        pl.kernel,
        mesh=plsc.VectorSubcoreMesh(core_axis_name="c", subcore_axis_name="s"),
        out_type=jax.ShapeDtypeStruct(
            (rows_per_worker * NUM_WORKERS // CHUNK, CHUNK, EMBED), jnp.float32
        ),
        scratch_types=[
            pltpu.VMEM((num_chunks, CHUNK), jnp.int32),
            pltpu.VMEM((CHUNK, EMBED), jnp.float32),
            pltpu.VMEM((CHUNK, EMBED), jnp.float32),
            pltpu.VMEM((CHUNK, EMBED), jnp.float32),
            pltpu.SemaphoreType.DMA,
            pltpu.SemaphoreType.DMA,
        ],
        compiler_params=pltpu.CompilerParams(use_tc_tiling_on_sc=False),
    )
    def emb_kernel(x_hbm, pe_hbm, table_hbm, out_hbm,
                   idx_v, pe_v, buf0, buf1, sem0, sem1):
        w = lax.axis_index("s") * NUM_CORES + lax.axis_index("c")
        chunk_base = w * num_chunks
        pltpu.sync_copy(x_hbm.at[w], idx_v)
        pltpu.sync_copy(pe_hbm, pe_v)
        pltpu.async_copy(table_hbm.at[idx_v.at[0]], buf0, sem0)
        pltpu.async_copy(table_hbm.at[idx_v.at[1]], buf1, sem1)

        def process(j, buf, sem):
            pltpu.make_async_copy(table_hbm.at[idx_v.at[j]], buf, sem).wait()

            def add_rows(i, carry):
                r0 = i * 4
                for k in range(4):
                    for d in range(VREGS_PER_ROW):
                        sl = pl.ds(d * LANES, LANES)
                        buf[r0 + k, sl] = buf[r0 + k, sl] + pe_v[r0 + k, sl]
                return carry

            lax.fori_loop(0, CHUNK // 4, add_rows, 0)
            pltpu.sync_copy(buf, out_hbm.at[chunk_base + j])

            @pl.when(j + 2 < num_chunks)
            def _():
                pltpu.async_copy(table_hbm.at[idx_v.at[j + 2]], buf, sem)

        def step(t, carry):
            process(2 * t, buf0, sem0)
            process(2 * t + 1, buf1, sem1)
            return carry

        lax.fori_loop(0, num_chunks // 2, step, 0)

    return emb_kernel


def kernel(x, table):
    batch, seq = x.shape
    vocab, embed = table.shape
    assert embed == EMBED and seq == SEQ
    rows = batch * seq
    assert rows % (NUM_WORKERS * CHUNK) == 0
    rows_per_worker = rows // NUM_WORKERS
    num_chunks = rows_per_worker // CHUNK

    x_split = x.reshape(NUM_WORKERS, num_chunks, CHUNK)
    pe = np.tile(_positional_encoding(SEQ, EMBED), (CHUNK // SEQ, 1))
    out = _build(rows_per_worker, vocab)(x_split, jnp.asarray(pe), table)
    return out.reshape(batch, seq, EMBED)
